# baseline (speedup 1.0000x reference)
import jax
import jax.numpy as jnp
from jax.experimental import pallas as pl
from jax.experimental.pallas import tpu as pltpu

_D = 256
_BR = 2304


def _vq_body(x_ref, w_ref, q_ref, loss_ref, perp_ref, idx_ref, acc_ref):
    i = pl.program_id(0)
    nsteps = pl.num_programs(0)
    w0 = w_ref[0:1, :]
    x = x_ref[...]
    d = w0 - x
    q_ref[...] = jnp.broadcast_to(w0, (_BR, _D))
    part = jnp.sum(d * d)

    @pl.when(i == 0)
    def _init():
        acc_ref[0, 0] = part
        idx_ref[...] = jnp.zeros_like(idx_ref)
        perp = jnp.exp(-(jnp.log(jnp.float32(1.0) + jnp.float32(1e-10))))
        perp_ref[...] = jnp.full((1, 1), perp, jnp.float32)

    @pl.when(i > 0)
    def _acc():
        acc_ref[0, 0] += part

    @pl.when(i == nsteps - 1)
    def _fin():
        total = jnp.float32(nsteps * _BR * _D)
        loss = acc_ref[0, 0] * (jnp.float32(1.25) / total)
        loss_ref[...] = jnp.full((1, 1), loss, jnp.float32)


def kernel(inputs, W):
    shape = inputs.shape
    flat = inputs.reshape(-1, _D)
    n = flat.shape[0]
    grid = n // _BR

    q, loss, perp, idx = pl.pallas_call(
        _vq_body,
        grid=(grid,),
        in_specs=[
            pl.BlockSpec((_BR, _D), lambda i: (i, 0)),
            pl.BlockSpec((8, _D), lambda i: (0, 0)),
        ],
        out_specs=[
            pl.BlockSpec((_BR, _D), lambda i: (i, 0)),
            pl.BlockSpec((1, 1), lambda i: (0, 0)),
            pl.BlockSpec((1, 1), lambda i: (0, 0)),
            pl.BlockSpec(shape[:2], lambda i: (0, 0)),
        ],
        out_shape=[
            jax.ShapeDtypeStruct((n, _D), jnp.float32),
            jax.ShapeDtypeStruct((1, 1), jnp.float32),
            jax.ShapeDtypeStruct((1, 1), jnp.float32),
            jax.ShapeDtypeStruct(shape[:2], jnp.int32),
        ],
        scratch_shapes=[pltpu.SMEM((1, 1), jnp.float32)],
    )(flat, W)

    return (q.reshape(shape), loss.reshape(()), perp.reshape(()), idx)
